# baseline (speedup 1.0000x reference)
import jax
import jax.numpy as jnp
from jax import lax
from jax.experimental import pallas as pl
from jax.experimental.pallas import tpu as pltpu
from jax.experimental.pallas import tpu_sc as plsc

_ROWS = 16384
_COLS = 4096

_SC_ROWS = 6144
_TC_ROWS = _ROWS - _SC_ROWS

_BM1 = 1024
_BM2 = 512

_NC = 2
_NS = 16
_NW = _NC * _NS
_SC_BASE = _TC_ROWS * _COLS
_PER_W = _SC_ROWS * _COLS // _NW
_CHUNK = 32768
_NCHUNK = _PER_W // _CHUNK


def _tc_minmax_body(x_ref, mn_ref, mx_ref):
    i = pl.program_id(0)
    bmn = jnp.min(x_ref[...])
    bmx = jnp.max(x_ref[...])

    @pl.when(i == 0)
    def _init():
        mn_ref[0, 0] = bmn
        mx_ref[0, 0] = bmx

    @pl.when(i > 0)
    def _acc():
        mn_ref[0, 0] = jnp.minimum(mn_ref[0, 0], bmn)
        mx_ref[0, 0] = jnp.maximum(mx_ref[0, 0], bmx)


def _sc_minmax_body(x_hbm, mn_hbm, mx_hbm, buf0, buf1, acc_mn, acc_mx,
                    sem0, sem1):
    cidx = lax.axis_index("c")
    sidx = lax.axis_index("s")
    wid = sidx * _NC + cidx
    base = _SC_BASE + wid * _PER_W

    bufs = (buf0, buf1)
    sems = (sem0, sem1)

    pltpu.async_copy(x_hbm.at[pl.ds(base, _CHUNK)], buf0, sem0)
    pltpu.async_copy(x_hbm.at[pl.ds(base + _CHUNK, _CHUNK)], buf1, sem1)

    _LANES = 8
    mns = tuple(jnp.full((16,), jnp.inf, jnp.float32) for _ in range(_LANES))
    mxs = tuple(jnp.full((16,), -jnp.inf, jnp.float32) for _ in range(_LANES))
    for k in range(_NCHUNK):
        b = k % 2
        buf = bufs[b]
        pltpu.make_async_copy(
            x_hbm.at[pl.ds(base + k * _CHUNK, _CHUNK)], buf, sems[b]
        ).wait()

        def inner(i, carry, buf=buf):
            cm, cM = carry
            off = i * (16 * _LANES)
            nm, nM = [], []
            for j in range(_LANES):
                v = buf[pl.ds(off + j * 16, 16)]
                nm.append(jnp.minimum(cm[j], v))
                nM.append(jnp.maximum(cM[j], v))
            return tuple(nm), tuple(nM)

        mns, mxs = lax.fori_loop(
            0, _CHUNK // (16 * _LANES), inner, (mns, mxs), unroll=2)
        if k + 2 < _NCHUNK:
            pltpu.async_copy(
                x_hbm.at[pl.ds(base + (k + 2) * _CHUNK, _CHUNK)],
                buf, sems[b])

    mn = mns[0]
    mx = mxs[0]
    for j in range(1, _LANES):
        mn = jnp.minimum(mn, mns[j])
        mx = jnp.maximum(mx, mxs[j])
    acc_mn[...] = mn
    acc_mx[...] = mx
    pltpu.sync_copy(acc_mn, mn_hbm.at[wid])
    pltpu.sync_copy(acc_mx, mx_hbm.at[wid])


_sc_minmax = pl.kernel(
    _sc_minmax_body,
    out_type=[
        jax.ShapeDtypeStruct((_NW, 16), jnp.float32),
        jax.ShapeDtypeStruct((_NW, 16), jnp.float32),
    ],
    mesh=plsc.VectorSubcoreMesh(core_axis_name="c", subcore_axis_name="s"),
    scratch_types=[
        pltpu.VMEM((_CHUNK,), jnp.float32),
        pltpu.VMEM((_CHUNK,), jnp.float32),
        pltpu.VMEM((16,), jnp.float32),
        pltpu.VMEM((16,), jnp.float32),
        pltpu.SemaphoreType.DMA,
        pltpu.SemaphoreType.DMA,
    ],
)


def _rescale_body(mn_ref, mx_ref, sc_mn_ref, sc_mx_ref, x_ref, o_ref):
    mn = jnp.minimum(mn_ref[0, 0], jnp.min(sc_mn_ref[...]))
    mx = jnp.maximum(mx_ref[0, 0], jnp.max(sc_mx_ref[...]))
    scale = 1.0 / (mx - mn)
    o_ref[...] = (x_ref[...] - mn) * scale


def kernel(inp):
    tc_mn, tc_mx = pl.pallas_call(
        _tc_minmax_body,
        grid=(_TC_ROWS // _BM1,),
        in_specs=[pl.BlockSpec((_BM1, _COLS), lambda i: (i, 0))],
        out_specs=[
            pl.BlockSpec((1, 1), lambda i: (0, 0), memory_space=pltpu.SMEM),
            pl.BlockSpec((1, 1), lambda i: (0, 0), memory_space=pltpu.SMEM),
        ],
        out_shape=[
            jax.ShapeDtypeStruct((1, 1), jnp.float32),
            jax.ShapeDtypeStruct((1, 1), jnp.float32),
        ],
    )(inp)

    sc_mn, sc_mx = _sc_minmax(inp.reshape(-1))

    out = pl.pallas_call(
        _rescale_body,
        grid=(_ROWS // _BM2,),
        in_specs=[
            pl.BlockSpec(memory_space=pltpu.SMEM),
            pl.BlockSpec(memory_space=pltpu.SMEM),
            pl.BlockSpec((_NW, 16), lambda i: (0, 0)),
            pl.BlockSpec((_NW, 16), lambda i: (0, 0)),
            pl.BlockSpec((_BM2, _COLS), lambda i: (i, 0)),
        ],
        out_specs=pl.BlockSpec((_BM2, _COLS), lambda i: (i, 0)),
        out_shape=jax.ShapeDtypeStruct((_ROWS, _COLS), jnp.float32),
    )(tc_mn, tc_mx, sc_mn, sc_mx, inp)
    return out


# speedup vs baseline: 1.6833x; 1.6833x over previous
import jax
import jax.numpy as jnp
from jax import lax
from jax.experimental import pallas as pl
from jax.experimental.pallas import tpu as pltpu
from jax.experimental.pallas import tpu_sc as plsc

_ROWS = 16384
_COLS = 4096

_SC_ROWS = 5120
_TC_ROWS = _ROWS - _SC_ROWS

_BM1 = 1024
_BM2 = 512

_NC = 2
_NS = 16
_NW = _NC * _NS
_ROWS_W = _SC_ROWS // _NW
_CH_ROWS = 8
_NCHUNK = _ROWS_W // _CH_ROWS


def _tc_minmax_body(x_ref, mn_ref, mx_ref):
    i = pl.program_id(0)
    bmn = jnp.min(x_ref[...])
    bmx = jnp.max(x_ref[...])

    @pl.when(i == 0)
    def _init():
        mn_ref[0, 0] = bmn
        mx_ref[0, 0] = bmx

    @pl.when(i > 0)
    def _acc():
        mn_ref[0, 0] = jnp.minimum(mn_ref[0, 0], bmn)
        mx_ref[0, 0] = jnp.maximum(mx_ref[0, 0], bmx)


def _sc_minmax_body(x_hbm, mn_hbm, mx_hbm, buf0, buf1, acc_mn, acc_mx,
                    sem0, sem1):
    cidx = lax.axis_index("c")
    sidx = lax.axis_index("s")
    wid = sidx * _NC + cidx
    row0 = _TC_ROWS + wid * _ROWS_W

    bufs = (buf0, buf1)
    sems = (sem0, sem1)

    pltpu.async_copy(x_hbm.at[pl.ds(row0, _CH_ROWS)], buf0, sem0)
    pltpu.async_copy(x_hbm.at[pl.ds(row0 + _CH_ROWS, _CH_ROWS)], buf1, sem1)

    mns = tuple(jnp.full((16,), jnp.inf, jnp.float32)
                for _ in range(_CH_ROWS))
    mxs = tuple(jnp.full((16,), -jnp.inf, jnp.float32)
                for _ in range(_CH_ROWS))
    for k in range(_NCHUNK):
        b = k % 2
        buf = bufs[b]
        pltpu.make_async_copy(
            x_hbm.at[pl.ds(row0 + k * _CH_ROWS, _CH_ROWS)], buf, sems[b]
        ).wait()

        def inner(i, carry, buf=buf):
            cm, cM = carry
            nm, nM = [], []
            for r in range(_CH_ROWS):
                v = buf[r, pl.ds(i * 16, 16)]
                nm.append(jnp.minimum(cm[r], v))
                nM.append(jnp.maximum(cM[r], v))
            return tuple(nm), tuple(nM)

        mns, mxs = lax.fori_loop(0, _COLS // 16, inner, (mns, mxs),
                                 unroll=2)
        if k + 2 < _NCHUNK:
            pltpu.async_copy(
                x_hbm.at[pl.ds(row0 + (k + 2) * _CH_ROWS, _CH_ROWS)],
                buf, sems[b])

    mn = mns[0]
    mx = mxs[0]
    for j in range(1, _CH_ROWS):
        mn = jnp.minimum(mn, mns[j])
        mx = jnp.maximum(mx, mxs[j])
    acc_mn[...] = mn
    acc_mx[...] = mx
    pltpu.sync_copy(acc_mn, mn_hbm.at[wid])
    pltpu.sync_copy(acc_mx, mx_hbm.at[wid])


_sc_minmax = pl.kernel(
    _sc_minmax_body,
    out_type=[
        jax.ShapeDtypeStruct((_NW, 16), jnp.float32),
        jax.ShapeDtypeStruct((_NW, 16), jnp.float32),
    ],
    mesh=plsc.VectorSubcoreMesh(core_axis_name="c", subcore_axis_name="s"),
    scratch_types=[
        pltpu.VMEM((_CH_ROWS, _COLS), jnp.float32),
        pltpu.VMEM((_CH_ROWS, _COLS), jnp.float32),
        pltpu.VMEM((16,), jnp.float32),
        pltpu.VMEM((16,), jnp.float32),
        pltpu.SemaphoreType.DMA,
        pltpu.SemaphoreType.DMA,
    ],
)


def _rescale_body(mn_ref, mx_ref, sc_mn_ref, sc_mx_ref, x_ref, o_ref):
    mn = jnp.minimum(mn_ref[0, 0], jnp.min(sc_mn_ref[...]))
    mx = jnp.maximum(mx_ref[0, 0], jnp.max(sc_mx_ref[...]))
    scale = 1.0 / (mx - mn)
    o_ref[...] = (x_ref[...] - mn) * scale


def kernel(inp):
    tc_mn, tc_mx = pl.pallas_call(
        _tc_minmax_body,
        grid=(_TC_ROWS // _BM1,),
        in_specs=[pl.BlockSpec((_BM1, _COLS), lambda i: (i, 0))],
        out_specs=[
            pl.BlockSpec((1, 1), lambda i: (0, 0), memory_space=pltpu.SMEM),
            pl.BlockSpec((1, 1), lambda i: (0, 0), memory_space=pltpu.SMEM),
        ],
        out_shape=[
            jax.ShapeDtypeStruct((1, 1), jnp.float32),
            jax.ShapeDtypeStruct((1, 1), jnp.float32),
        ],
    )(inp)

    sc_mn, sc_mx = _sc_minmax(inp)

    out = pl.pallas_call(
        _rescale_body,
        grid=(_ROWS // _BM2,),
        in_specs=[
            pl.BlockSpec(memory_space=pltpu.SMEM),
            pl.BlockSpec(memory_space=pltpu.SMEM),
            pl.BlockSpec((_NW, 16), lambda i: (0, 0)),
            pl.BlockSpec((_NW, 16), lambda i: (0, 0)),
            pl.BlockSpec((_BM2, _COLS), lambda i: (i, 0)),
        ],
        out_specs=pl.BlockSpec((_BM2, _COLS), lambda i: (i, 0)),
        out_shape=jax.ShapeDtypeStruct((_ROWS, _COLS), jnp.float32),
    )(tc_mn, tc_mx, sc_mn, sc_mx, inp)
    return out


# speedup vs baseline: 1.7534x; 1.0417x over previous
import jax
import jax.numpy as jnp
from jax.experimental import pallas as pl
from jax.experimental.pallas import tpu as pltpu

_ROWS = 16384
_COLS = 4096
_BM = 512
_NBLK = _ROWS // _BM


def _body(x_ref, o_ref, acc_ref):
    p = pl.program_id(0)
    i = pl.program_id(1)

    @pl.when(p == 0)
    def _reduce():
        bmn = jnp.min(x_ref[...])
        bmx = jnp.max(x_ref[...])

        @pl.when(i == 0)
        def _init():
            acc_ref[0] = bmn
            acc_ref[1] = bmx

        @pl.when(i > 0)
        def _acc():
            acc_ref[0] = jnp.minimum(acc_ref[0], bmn)
            acc_ref[1] = jnp.maximum(acc_ref[1], bmx)

    @pl.when(p == 1)
    def _rescale():
        mn = acc_ref[0]
        scale = 1.0 / (acc_ref[1] - mn)
        o_ref[...] = (x_ref[...] - mn) * scale


def kernel(inp):
    return pl.pallas_call(
        _body,
        grid=(2, _NBLK),
        in_specs=[pl.BlockSpec((_BM, _COLS), lambda p, i: (i, 0))],
        out_specs=pl.BlockSpec((_BM, _COLS), lambda p, i: (i * p, 0)),
        out_shape=jax.ShapeDtypeStruct((_ROWS, _COLS), jnp.float32),
        scratch_shapes=[pltpu.SMEM((2,), jnp.float32)],
    )(inp)
